# baseline (speedup 1.0000x reference)
import functools

import jax
import jax.numpy as jnp
from jax import lax
from jax.experimental import pallas as pl
from jax.experimental.pallas import tpu as pltpu
from jax.experimental.pallas import tpu_sc as plsc

VOCAB = 49408
D = 512
S = 77
B = 4096
NC = 2
NS = 16
NW = NC * NS
WB = B // NW
LANES = 16


def kernel(input_ids, token_embedding, position_embedding):
    ids_t = input_ids.astype(jnp.int32).T.reshape(-1)
    mesh = plsc.VectorSubcoreMesh(core_axis_name="c", subcore_axis_name="s")

    @functools.partial(
        pl.kernel,
        out_type=jax.ShapeDtypeStruct((B, S * D), jnp.float32),
        mesh=mesh,
        scratch_types=[
            pltpu.VMEM((WB,), jnp.int32),
            pltpu.VMEM((WB, D), jnp.float32),
            pltpu.VMEM((1, D), jnp.float32),
            pltpu.SemaphoreType.DMA,
        ],
    )
    def gather_add(ids_hbm, tab_hbm, pos_hbm, out_hbm, idx_v, rows_v, posr_v, sem):
        wid = lax.axis_index("s") * NC + lax.axis_index("c")
        b0 = wid * WB

        @pl.loop(0, S)
        def _(s):
            pltpu.sync_copy(ids_hbm.at[pl.ds(s * B + b0, WB)], idx_v)
            pltpu.async_copy(tab_hbm.at[idx_v], rows_v, sem).wait()
            pltpu.sync_copy(pos_hbm.at[pl.ds(s, 1)], posr_v)
            for c in range(D // LANES):
                pc = posr_v[0, pl.ds(c * LANES, LANES)]

                @pl.loop(0, WB)
                def _(r):
                    plsc.addupdate(rows_v.at[r, pl.ds(c * LANES, LANES)], pc)

            pltpu.sync_copy(rows_v, out_hbm.at[pl.ds(b0, WB), pl.ds(s * D, D)])

    out = gather_add(ids_t, token_embedding, position_embedding)
    return out.reshape(B, S, D)


# speedup vs baseline: 1.4284x; 1.4284x over previous
import functools

import jax
import jax.numpy as jnp
from jax import lax
from jax.experimental import pallas as pl
from jax.experimental.pallas import tpu as pltpu
from jax.experimental.pallas import tpu_sc as plsc

VOCAB = 49408
D = 512
S = 77
B = 4096
NC = 2
NS = 16
NW = NC * NS
STRIPE = B // NW
WB = 64
NWIN = S * 2
LANES = 16


def kernel(input_ids, token_embedding, position_embedding):
    ids_t = input_ids.astype(jnp.int32).T
    mesh = plsc.VectorSubcoreMesh(core_axis_name="c", subcore_axis_name="s")

    @functools.partial(
        pl.kernel,
        out_type=jax.ShapeDtypeStruct((B, S * D), jnp.float32),
        mesh=mesh,
        scratch_types=[
            pltpu.VMEM((S, STRIPE), jnp.int32),
            pltpu.VMEM((S, D), jnp.float32),
            pltpu.VMEM((WB, D), jnp.float32),
            pltpu.VMEM((WB, D), jnp.float32),
            pltpu.SemaphoreType.DMA,
            pltpu.SemaphoreType.DMA,
            pltpu.SemaphoreType.DMA,
            pltpu.SemaphoreType.DMA,
        ],
    )
    def gather_add(ids_hbm, tab_hbm, pos_hbm, out_hbm,
                   idx_v, pos_v, rows0_v, rows1_v,
                   gsem0, gsem1, osem0, osem1):
        wid = lax.axis_index("s") * NC + lax.axis_index("c")
        b0 = wid * STRIPE
        rows = (rows0_v, rows1_v)
        gsem = (gsem0, gsem1)
        osem = (osem0, osem1)

        pltpu.sync_copy(ids_hbm.at[:, pl.ds(b0, STRIPE)], idx_v)
        pltpu.sync_copy(pos_hbm, pos_v)

        def idx_slice(s, h):
            return idx_v.at[s, pl.ds(h * WB, WB)]

        def out_slice(s, h):
            return out_hbm.at[pl.ds(b0 + h * WB, WB), pl.ds(s * D, D)]

        pltpu.async_copy(tab_hbm.at[idx_slice(0, 0)], rows0_v, gsem0)

        @pl.loop(0, NWIN, step=2)
        def _(w):
            for boff in range(2):
                ww = w + boff
                bsel = boff
                s = ww >> 1
                h = ww & 1

                @pl.when(ww > 0)
                def _():
                    pltpu.make_async_copy(
                        rows[1 - bsel], out_slice(s, h), osem[1 - bsel]
                    ).wait()

                nxt = ww + 1

                @pl.when(nxt < NWIN)
                def _():
                    pltpu.async_copy(
                        tab_hbm.at[idx_slice(nxt >> 1, nxt & 1)],
                        rows[1 - bsel],
                        gsem[1 - bsel],
                    )

                pltpu.make_async_copy(
                    tab_hbm.at[idx_slice(s, h)], rows[bsel], gsem[bsel]
                ).wait()
                for c in range(D // LANES):
                    pc = pos_v[s, pl.ds(c * LANES, LANES)]

                    @pl.loop(0, WB, unroll=8)
                    def _(r):
                        plsc.addupdate(rows[bsel].at[r, pl.ds(c * LANES, LANES)], pc)

                pltpu.async_copy(rows[bsel], out_slice(s, h), osem[bsel])

        pltpu.make_async_copy(rows1_v, out_slice(S - 1, 1), osem1).wait()

    out = gather_add(ids_t, token_embedding, position_embedding)
    return out.reshape(B, S, D)
